# baseline (speedup 1.0000x reference)
import functools

import jax
import jax.numpy as jnp
from jax import lax
from jax.experimental import pallas as pl
from jax.experimental.pallas import tpu as pltpu
from jax.experimental.pallas import tpu_sc as plsc

_H = 200
_BT = 128
_DIM = 64
_NW = 32
_NSLOT = 4
_NBLK = (_H * _BT) // _NW


def _emb_body(idx_hbm, table_hbm, out_hbm, idx_v, gb, tb, isem, gsem, osem):
    wid = lax.axis_index("s") * 2 + lax.axis_index("c")
    blk0 = wid * _NBLK
    rows = [lax.iota(jnp.int32, 16) + 16 * k for k in range(8)]

    def hb(g):
        blk = blk0 + g
        return blk // _BT, blk % _BT

    def i_copy(g, s):
        h, bt = hb(g)
        return pltpu.make_async_copy(
            idx_hbm.at[h].at[pl.ds(bt * 128, 128)], idx_v.at[s], isem.at[s])

    def g_copy(s):
        return pltpu.make_async_copy(
            table_hbm.at[idx_v.at[s]], gb.at[s], gsem.at[s])

    def o_copy(g, s, dh):
        h, bt = hb(g)
        return pltpu.make_async_copy(
            tb.at[s].at[dh], out_hbm.at[h, dh, bt], osem.at[s])

    def transpose(s):
        gbs = gb.at[s]

        def dbody(d8, carry):
            for dd in range(8):
                cols = jnp.full((16,), d8 * 8 + dd, jnp.int32)
                for b16 in range(8):
                    v = plsc.load_gather(gbs, [rows[b16], cols])
                    tb[s, d8, dd, pl.ds(b16 * 16, 16)] = v
            return carry

        lax.fori_loop(0, 8, dbody, 0)

    i_copy(0, 0).start()
    i_copy(1, 1).start()
    i_copy(0, 0).wait()
    g_copy(0).start()

    def body(g, carry):
        s = lax.rem(g, _NSLOT)
        s1 = lax.rem(g + 1, _NSLOT)
        s2 = lax.rem(g + 2, _NSLOT)

        @pl.when(g < _NBLK - 1)
        def _():
            i_copy(g + 1, s1).wait()
            g_copy(s1).start()

        @pl.when(g < _NBLK - 2)
        def _():
            i_copy(g + 2, s2).start()

        g_copy(s).wait()

        @pl.when(g >= _NSLOT)
        def _():
            for dh in range(8):
                o_copy(g - _NSLOT, s, dh).wait()

        transpose(s)
        for dh in range(8):
            o_copy(g, s, dh).start()
        return carry

    lax.fori_loop(0, _NBLK, body, 0)

    for s in range(_NSLOT):
        for dh in range(8):
            o_copy(_NBLK - _NSLOT + s, s, dh).wait()


def kernel(token_ids, weight):
    tid_t = jnp.transpose(token_ids)

    mesh = plsc.VectorSubcoreMesh(core_axis_name="c", subcore_axis_name="s")
    emb = functools.partial(
        pl.kernel,
        mesh=mesh,
        out_type=jax.ShapeDtypeStruct((_H, 8, _BT, 8, 128), jnp.float32),
        scratch_types=[
            pltpu.VMEM((_NSLOT, 128), jnp.int32),
            pltpu.VMEM((_NSLOT, 128, _DIM), jnp.float32),
            pltpu.VMEM((_NSLOT, 8, 8, 128), jnp.float32),
            pltpu.SemaphoreType.DMA((_NSLOT,)),
            pltpu.SemaphoreType.DMA((_NSLOT,)),
            pltpu.SemaphoreType.DMA((_NSLOT,)),
        ],
        compiler_params=pltpu.CompilerParams(
            use_tc_tiling_on_sc=False, needs_layout_passes=False),
    )(_emb_body)
    out5 = emb(tid_t, weight)
    return out5.transpose(2, 4, 0, 1, 3).reshape(16384, _H, _DIM)


# speedup vs baseline: 1.6105x; 1.6105x over previous
import functools

import jax
import jax.numpy as jnp
from jax import lax
from jax.experimental import pallas as pl
from jax.experimental.pallas import tpu as pltpu
from jax.experimental.pallas import tpu_sc as plsc

_H = 200
_BT = 128
_DIM = 64
_NW = 32
_NSLOT = 4
_NBLK = (_H * _BT) // _NW


def _emb_body(idx_hbm, table_hbm, out_hbm, idx_v, gb, tb, isem, gsem, osem):
    wid = lax.axis_index("s") * 2 + lax.axis_index("c")
    blk0 = wid * _NBLK
    rows = [lax.iota(jnp.int32, 16) + 16 * k for k in range(8)]

    def hb(g):
        blk = blk0 + g
        return blk // _BT, blk % _BT

    def i_copy(g, s):
        h, bt = hb(g)
        return pltpu.make_async_copy(
            idx_hbm.at[h].at[pl.ds(bt * 128, 128)], idx_v.at[s], isem.at[s])

    def g_copy(s):
        return pltpu.make_async_copy(
            table_hbm.at[idx_v.at[s]], gb.at[s], gsem.at[s])

    def o_copy(g, s, dh):
        h, bt = hb(g)
        return pltpu.make_async_copy(
            tb.at[s].at[dh], out_hbm.at[h, dh, bt], osem.at[s])

    def transpose(s):
        gbs = gb.at[s]

        @plsc.parallel_loop(0, 8, 1, unroll=2)
        def _(d8):
            for dd in range(8):
                cols = jnp.full((16,), d8 * 8 + dd, jnp.int32)
                for b16 in range(8):
                    v = plsc.load_gather(gbs, [rows[b16], cols])
                    tb[s, d8, dd, pl.ds(b16 * 16, 16)] = v

    i_copy(0, 0).start()
    i_copy(1, 1).start()
    i_copy(0, 0).wait()
    g_copy(0).start()

    def body(g, carry):
        s = lax.rem(g, _NSLOT)
        s1 = lax.rem(g + 1, _NSLOT)
        s2 = lax.rem(g + 2, _NSLOT)

        @pl.when(g < _NBLK - 1)
        def _():
            i_copy(g + 1, s1).wait()
            g_copy(s1).start()

        @pl.when(g < _NBLK - 2)
        def _():
            i_copy(g + 2, s2).start()

        g_copy(s).wait()

        @pl.when(g >= _NSLOT)
        def _():
            for dh in range(8):
                o_copy(g - _NSLOT, s, dh).wait()

        transpose(s)
        for dh in range(8):
            o_copy(g, s, dh).start()
        return carry

    lax.fori_loop(0, _NBLK, body, 0)

    for s in range(_NSLOT):
        for dh in range(8):
            o_copy(_NBLK - _NSLOT + s, s, dh).wait()


def kernel(token_ids, weight):
    tid_t = jnp.transpose(token_ids)

    mesh = plsc.VectorSubcoreMesh(core_axis_name="c", subcore_axis_name="s")
    emb = functools.partial(
        pl.kernel,
        mesh=mesh,
        out_type=jax.ShapeDtypeStruct((_H, 8, _BT, 8, 128), jnp.float32),
        scratch_types=[
            pltpu.VMEM((_NSLOT, 128), jnp.int32),
            pltpu.VMEM((_NSLOT, 128, _DIM), jnp.float32),
            pltpu.VMEM((_NSLOT, 8, 8, 128), jnp.float32),
            pltpu.SemaphoreType.DMA((_NSLOT,)),
            pltpu.SemaphoreType.DMA((_NSLOT,)),
            pltpu.SemaphoreType.DMA((_NSLOT,)),
        ],
        compiler_params=pltpu.CompilerParams(
            use_tc_tiling_on_sc=False, needs_layout_passes=False),
    )(_emb_body)
    out5 = emb(tid_t, weight)
    return out5.transpose(2, 4, 0, 1, 3).reshape(16384, _H, _DIM)


# speedup vs baseline: 4.1766x; 2.5933x over previous
import functools

import jax
import jax.numpy as jnp
from jax import lax
from jax.experimental import pallas as pl
from jax.experimental.pallas import tpu as pltpu
from jax.experimental.pallas import tpu_sc as plsc

_H = 200
_BT = 128
_DIM = 64
_NW = 32
_NSLOT = 4
_NBLK = (_H * _BT) // _NW
_TP = 136


def _emb_body(idx_hbm, table_hbm, out_hbm, idx_v, gb, tb, isem, gsem, osem):
    wid = lax.axis_index("s") * 2 + lax.axis_index("c")
    blk0 = wid * _NBLK
    iota = lax.iota(jnp.int32, 16)
    dhv = [(iota + 16 * c) // 8 for c in range(4)]
    dlv = [(iota + 16 * c) % 8 for c in range(4)]

    def hb(g):
        blk = blk0 + g
        return blk // _BT, blk % _BT

    def i_copy(g, s):
        h, bt = hb(g)
        return pltpu.make_async_copy(
            idx_hbm.at[h].at[pl.ds(bt * 128, 128)], idx_v.at[s], isem.at[s])

    def g_copy(s):
        return pltpu.make_async_copy(
            table_hbm.at[idx_v.at[s]], gb.at[s], gsem.at[s])

    def o_copy(g, s):
        h, bt = hb(g)
        return pltpu.make_async_copy(
            tb.at[s].at[:, :, pl.ds(0, 128)], out_hbm.at[h].at[:, bt],
            osem.at[s])

    def transpose(s):
        @plsc.parallel_loop(0, 128, 1, unroll=2)
        def _(b):
            bb = jnp.full((16,), b, jnp.int32)
            for c in range(4):
                v = gb[s, b, pl.ds(16 * c, 16)]
                plsc.store_scatter(tb.at[s], [dhv[c], dlv[c], bb], v)

    i_copy(0, 0).start()
    i_copy(1, 1).start()
    i_copy(0, 0).wait()
    g_copy(0).start()

    def body(g, carry):
        s = lax.rem(g, _NSLOT)
        s1 = lax.rem(g + 1, _NSLOT)
        s2 = lax.rem(g + 2, _NSLOT)

        @pl.when(g < _NBLK - 1)
        def _():
            i_copy(g + 1, s1).wait()
            g_copy(s1).start()

        @pl.when(g < _NBLK - 2)
        def _():
            i_copy(g + 2, s2).start()

        g_copy(s).wait()

        @pl.when(g >= _NSLOT)
        def _():
            o_copy(g - _NSLOT, s).wait()

        transpose(s)
        o_copy(g, s).start()
        return carry

    lax.fori_loop(0, _NBLK, body, 0)

    for s in range(_NSLOT):
        o_copy(_NBLK - _NSLOT + s, s).wait()


def kernel(token_ids, weight):
    tid_t = jnp.transpose(token_ids)

    mesh = plsc.VectorSubcoreMesh(core_axis_name="c", subcore_axis_name="s")
    emb = functools.partial(
        pl.kernel,
        mesh=mesh,
        out_type=jax.ShapeDtypeStruct((_H, 8, _BT, 8, 128), jnp.float32),
        scratch_types=[
            pltpu.VMEM((_NSLOT, 128), jnp.int32),
            pltpu.VMEM((_NSLOT, 128, _DIM), jnp.float32),
            pltpu.VMEM((_NSLOT, 8, 8, _TP), jnp.float32),
            pltpu.SemaphoreType.DMA((_NSLOT,)),
            pltpu.SemaphoreType.DMA((_NSLOT,)),
            pltpu.SemaphoreType.DMA((_NSLOT,)),
        ],
        compiler_params=pltpu.CompilerParams(
            use_tc_tiling_on_sc=False, needs_layout_passes=False),
    )(_emb_body)
    out5 = emb(tid_t, weight)
    return out5.transpose(2, 4, 0, 1, 3).reshape(16384, _H, _DIM)


# speedup vs baseline: 4.6363x; 1.1101x over previous
import functools

import jax
import jax.numpy as jnp
from jax import lax
from jax.experimental import pallas as pl
from jax.experimental.pallas import tpu as pltpu
from jax.experimental.pallas import tpu_sc as plsc

_H = 200
_BT = 128
_DIM = 64
_NW = 32
_NSLOT = 6
_NBLK = (_H * _BT) // _NW
_TP = 136


def _emb_body(idx_hbm, table_hbm, out_hbm, idx_v, gb, tb, isem, gsem, osem):
    wid = lax.axis_index("s") * 2 + lax.axis_index("c")
    blk0 = wid * _NBLK
    iota = lax.iota(jnp.int32, 16)
    dhv = [(iota + 16 * c) // 8 for c in range(4)]
    dlv = [(iota + 16 * c) % 8 for c in range(4)]

    def hb(g):
        blk = blk0 + g
        return blk // _BT, blk % _BT

    def i_copy(g, s):
        h, bt = hb(g)
        return pltpu.make_async_copy(
            idx_hbm.at[h].at[pl.ds(bt * 128, 128)], idx_v.at[s], isem.at[s])

    def g_copy(s):
        return pltpu.make_async_copy(
            table_hbm.at[idx_v.at[s]], gb.at[s], gsem.at[s])

    def o_copy(g, s):
        h, bt = hb(g)
        return pltpu.make_async_copy(
            tb.at[s].at[:, :, pl.ds(0, 128)], out_hbm.at[h].at[:, bt],
            osem.at[s])

    def transpose(s):
        @plsc.parallel_loop(0, 128, 1, unroll=2)
        def _(b):
            bb = jnp.full((16,), b, jnp.int32)
            for c in range(4):
                v = gb[s, b, pl.ds(16 * c, 16)]
                plsc.store_scatter(tb.at[s], [dhv[c], dlv[c], bb], v)

    for k in range(4):
        i_copy(k, k).start()
    i_copy(0, 0).wait()
    g_copy(0).start()
    i_copy(1, 1).wait()
    g_copy(1).start()

    def body(g, carry):
        s = lax.rem(g, _NSLOT)
        s2 = lax.rem(g + 2, _NSLOT)
        s4 = lax.rem(g + 4, _NSLOT)

        @pl.when(g < _NBLK - 2)
        def _():
            i_copy(g + 2, s2).wait()
            g_copy(s2).start()

        @pl.when(g < _NBLK - 4)
        def _():
            i_copy(g + 4, s4).start()

        g_copy(s).wait()

        @pl.when(g >= _NSLOT)
        def _():
            o_copy(g - _NSLOT, s).wait()

        transpose(s)
        o_copy(g, s).start()
        return carry

    lax.fori_loop(0, _NBLK, body, 0)

    for k in range(_NSLOT):
        g = _NBLK - _NSLOT + k
        o_copy(g, g % _NSLOT).wait()


def kernel(token_ids, weight):
    tid_t = jnp.transpose(token_ids)

    mesh = plsc.VectorSubcoreMesh(core_axis_name="c", subcore_axis_name="s")
    emb = functools.partial(
        pl.kernel,
        mesh=mesh,
        out_type=jax.ShapeDtypeStruct((_H, 8, _BT, 8, 128), jnp.float32),
        scratch_types=[
            pltpu.VMEM((_NSLOT, 128), jnp.int32),
            pltpu.VMEM((_NSLOT, 128, _DIM), jnp.float32),
            pltpu.VMEM((_NSLOT, 8, 8, _TP), jnp.float32),
            pltpu.SemaphoreType.DMA((_NSLOT,)),
            pltpu.SemaphoreType.DMA((_NSLOT,)),
            pltpu.SemaphoreType.DMA((_NSLOT,)),
        ],
        compiler_params=pltpu.CompilerParams(
            use_tc_tiling_on_sc=False, needs_layout_passes=False),
    )(_emb_body)
    out5 = emb(tid_t, weight)
    return out5.transpose(2, 4, 0, 1, 3).reshape(16384, _H, _DIM)
